# baseline (speedup 1.0000x reference)
import functools

import jax
import jax.numpy as jnp
import numpy as np
from jax import lax
from jax.experimental import pallas as pl
from jax.experimental.pallas import tpu as pltpu
from jax.experimental.pallas import tpu_sc as plsc

H = 7
NC, NS = 2, 16
NW = NC * NS
CHUNK = 128
KG = 8
ROW = 8
BB = 32


def _sc_gather(ids3, table):
    nchunks = ids3.shape[1]
    per_w = nchunks * CHUNK
    grp = KG * CHUNK
    mesh = plsc.VectorSubcoreMesh(core_axis_name="c", subcore_axis_name="s")

    @functools.partial(
        pl.kernel,
        out_type=jax.ShapeDtypeStruct((NW * per_w, ROW), jnp.float32),
        mesh=mesh,
        scratch_types=[
            pltpu.VMEM((nchunks, CHUNK), jnp.int32),
            pltpu.VMEM((grp, ROW), jnp.float32),
            pltpu.SemaphoreType.DMA,
        ],
        compiler_params=pltpu.CompilerParams(use_tc_tiling_on_sc=False),
    )
    def gather_kernel(ids_hbm, table_hbm, out_hbm, idx_v, rows_v, sem):
        wid = lax.axis_index("s") * NC + lax.axis_index("c")
        pltpu.sync_copy(ids_hbm.at[wid], idx_v)

        def group(g, carry):
            copies = [
                pltpu.async_copy(
                    table_hbm.at[idx_v.at[g * KG + j]],
                    rows_v.at[pl.ds(j * CHUNK, CHUNK)], sem)
                for j in range(KG)
            ]
            for c in copies:
                c.wait()
            pltpu.sync_copy(rows_v, out_hbm.at[pl.ds(wid * per_w + g * grp, grp)])
            return carry

        lax.fori_loop(0, nchunks // KG, group, 0)

    return gather_kernel(ids3, table)


def _tc_expand(g, amp14, b, l):
    n = 2 * H

    def body(amp_ref, g_ref, of_ref, oa_ref, op_ref):
        g3 = g_ref[...].reshape(BB, l, ROW)
        fs = g3[:, :, 0:1]
        ff = g3[:, :, 1:2]
        ma = g3[:, :, 2:3]
        m1 = g3[:, :, 3:4]
        phi = g3[:, :, 4:5]
        k = lax.broadcasted_iota(jnp.int32, (BB, l, n), 2)
        sel = k < H
        hk = (k % H + 1).astype(jnp.float32)
        of_ref[...] = jnp.where(sel, fs, ff) * hk
        oa_ref[...] = jnp.where(sel, ma, m1) * amp_ref[...].reshape(1, 1, n)
        op_ref[...] = jnp.broadcast_to(phi, (BB, l, n))

    out3 = [jax.ShapeDtypeStruct((b, l, n), jnp.float32)] * 3
    return pl.pallas_call(
        body,
        grid=(b // BB,),
        in_specs=[pl.BlockSpec((1, n), lambda i: (0, 0)),
                  pl.BlockSpec((BB * l, ROW), lambda i: (i, 0))],
        out_specs=[pl.BlockSpec((BB, l, n), lambda i: (i, 0, 0))] * 3,
        out_shape=out3,
    )(amp14, g)


def kernel(ids, freq_slow, freq_fast, amplitudes, phase, scale_mix,
           decay_slow, decay_fast):
    B, L = ids.shape
    T = B * L
    mix = jax.nn.sigmoid(scale_mix)
    mix_a = mix * amplitudes
    m1_a = (1.0 - mix) * amplitudes
    z = jnp.zeros_like(freq_slow)
    table = jnp.stack(
        [freq_slow, freq_fast, mix_a, m1_a, phase, z, z, z], axis=1)

    nchunks = T // (NW * CHUNK)
    ids3 = ids.reshape(NW, nchunks, CHUNK)
    g = _sc_gather(ids3, table)

    h = jnp.arange(1, H + 1, dtype=jnp.float32)
    amp14 = jnp.concatenate(
        [1.0 / (h ** decay_slow), 1.0 / (h ** decay_fast)]).reshape(1, 2 * H)
    return _tc_expand(g, amp14, B, L)
